# baseline (speedup 1.0000x reference)
import functools

import jax
import jax.numpy as jnp
from jax import lax
from jax.experimental.layout import Layout, with_layout_constraint
from jax.experimental import pallas as pl
from jax.experimental.pallas import tpu as pltpu
from jax.experimental.pallas import tpu_sc as plsc

_NC = 2
_NS = 16
_NW = _NC * _NS

_L = 16
_BB = 128
_NBUF = 4
_TBUF = 2


@functools.lru_cache(maxsize=None)
def _make_lookup(bsz: int, seq: int, v: int, d: int):
    assert bsz == _NW * _BB and d % 8 == 0
    n_fb = d // 8
    mesh = plsc.VectorSubcoreMesh(core_axis_name="c", subcore_axis_name="s")

    @functools.partial(
        pl.kernel,
        mesh=mesh,
        compiler_params=pltpu.CompilerParams(
            use_tc_tiling_on_sc=False, needs_layout_passes=False
        ),
        out_type=jax.ShapeDtypeStruct((seq, n_fb, _NW, 8, _BB), jnp.float32),
        scratch_types=[
            pltpu.VMEM((seq, _BB), jnp.int32),
            [pltpu.VMEM((_BB, d), jnp.float32) for _ in range(_NBUF)],
            [pltpu.VMEM((d, _BB), jnp.float32) for _ in range(_TBUF)],
            [pltpu.SemaphoreType.DMA for _ in range(_NBUF)],
            [pltpu.SemaphoreType.DMA for _ in range(_TBUF)],
        ],
    )
    def lookup(table2, idx2, out5, idx_v, rows, tbufs, gsem, ssem):
        wid = lax.axis_index("s") * _NC + lax.axis_index("c")
        col = pl.multiple_of(wid * _BB, _BB)

        pltpu.sync_copy(idx2.at[:, pl.ds(col, _BB)], idx_v)

        def fire(l, rb):
            pltpu.async_copy(table2.at[idx_v.at[l]], rows[rb], gsem[rb])

        def wait_gather(rb):
            pltpu.make_async_copy(
                table2.at[idx_v.at[0]], rows[rb], gsem[rb]
            ).wait()

        def wait_stores(tb):
            for fb in range(n_fb):
                pltpu.make_async_copy(
                    tbufs[tb].at[pl.ds(8 * fb, 8)],
                    out5.at[0, fb, 0],
                    ssem[tb],
                ).wait()

        for l0 in range(_NBUF):
            fire(l0, l0)

        def step(g, carry):
            fidx = [h * _L + lax.iota(jnp.int32, _L) for h in range(d // _L)]
            for rb in range(_NBUF):
                tb = rb % _TBUF
                l = g * _NBUF + rb
                wait_gather(rb)

                @pl.when(l >= _TBUF)
                def _():
                    wait_stores(tb)

                @plsc.parallel_loop(0, _BB, unroll=16)
                def _(b):
                    bidx = jnp.full((_L,), b, jnp.int32)
                    for h in range(d // _L):
                        hseg = h * _L + lax.iota(jnp.int32, _L)
                        vals = rows[rb][b, pl.ds(h * _L, _L)]
                        plsc.store_scatter(tbufs[tb], [hseg, bidx], vals)

                @pl.when(l + _NBUF < seq)
                def _():
                    fire(l + _NBUF, rb)

                for fb in range(n_fb):
                    pltpu.async_copy(
                        tbufs[tb].at[pl.ds(8 * fb, 8)],
                        out5.at[l, fb, wid],
                        ssem[tb],
                    )

            return carry

        lax.fori_loop(0, seq // _NBUF, step, 0)

        for tb in range(_TBUF):
            wait_stores(tb)

    return lookup


def kernel(x, weight):
    bsz, seq = x.shape
    v, d = weight.shape
    out5 = _make_lookup(bsz, seq, v, d)(weight, x.T)
    return out5.transpose(2, 4, 0, 1, 3).reshape(bsz, seq, d)


# speedup vs baseline: 1.0019x; 1.0019x over previous
import functools

import jax
import jax.numpy as jnp
from jax import lax
from jax.experimental.layout import Layout, with_layout_constraint
from jax.experimental import pallas as pl
from jax.experimental.pallas import tpu as pltpu
from jax.experimental.pallas import tpu_sc as plsc

_NC = 2
_NS = 16
_NW = _NC * _NS

_L = 16
_BB = 128
_NBUF = 4
_TBUF = 4


@functools.lru_cache(maxsize=None)
def _make_lookup(bsz: int, seq: int, v: int, d: int):
    assert bsz == _NW * _BB and d % 8 == 0
    n_fb = d // 8
    mesh = plsc.VectorSubcoreMesh(core_axis_name="c", subcore_axis_name="s")

    @functools.partial(
        pl.kernel,
        mesh=mesh,
        compiler_params=pltpu.CompilerParams(
            use_tc_tiling_on_sc=False, needs_layout_passes=False
        ),
        out_type=jax.ShapeDtypeStruct((seq, n_fb, _NW, 8, _BB), jnp.float32),
        scratch_types=[
            pltpu.VMEM((seq, _BB), jnp.int32),
            [pltpu.VMEM((_BB, d), jnp.float32) for _ in range(_NBUF)],
            [pltpu.VMEM((d, _BB), jnp.float32) for _ in range(_TBUF)],
            [pltpu.SemaphoreType.DMA for _ in range(_NBUF)],
            [pltpu.SemaphoreType.DMA for _ in range(_TBUF)],
        ],
    )
    def lookup(table2, idx2, out5, idx_v, rows, tbufs, gsem, ssem):
        wid = lax.axis_index("s") * _NC + lax.axis_index("c")
        col = pl.multiple_of(wid * _BB, _BB)

        pltpu.sync_copy(idx2.at[:, pl.ds(col, _BB)], idx_v)

        def fire(l, rb):
            pltpu.async_copy(table2.at[idx_v.at[l]], rows[rb], gsem[rb])

        def wait_gather(rb):
            pltpu.make_async_copy(
                table2.at[idx_v.at[0]], rows[rb], gsem[rb]
            ).wait()

        def wait_stores(tb):
            for fb in range(n_fb):
                pltpu.make_async_copy(
                    tbufs[tb].at[pl.ds(8 * fb, 8)],
                    out5.at[0, fb, 0],
                    ssem[tb],
                ).wait()

        for l0 in range(_NBUF):
            fire(l0, l0)

        def step(g, carry):
            fidx = [h * _L + lax.iota(jnp.int32, _L) for h in range(d // _L)]
            for rb in range(_NBUF):
                tb = rb % _TBUF
                l = g * _NBUF + rb
                wait_gather(rb)

                @pl.when(l >= _TBUF)
                def _():
                    wait_stores(tb)

                @plsc.parallel_loop(0, _BB, unroll=8)
                def _(b):
                    bidx = jnp.full((_L,), b, jnp.int32)
                    for h in range(d // _L):
                        hseg = h * _L + lax.iota(jnp.int32, _L)
                        vals = rows[rb][b, pl.ds(h * _L, _L)]
                        plsc.store_scatter(tbufs[tb], [hseg, bidx], vals)

                @pl.when(l + _NBUF < seq)
                def _():
                    fire(l + _NBUF, rb)

                for fb in range(n_fb):
                    pltpu.async_copy(
                        tbufs[tb].at[pl.ds(8 * fb, 8)],
                        out5.at[l, fb, wid],
                        ssem[tb],
                    )

            return carry

        lax.fori_loop(0, seq // _NBUF, step, 0)

        for tb in range(_TBUF):
            wait_stores(tb)

    return lookup


def kernel(x, weight):
    bsz, seq = x.shape
    v, d = weight.shape
    out5 = _make_lookup(bsz, seq, v, d)(weight, x.T)
    return out5.transpose(2, 4, 0, 1, 3).reshape(bsz, seq, d)
